# baseline (speedup 1.0000x reference)
import functools

import jax
import jax.numpy as jnp
from jax import lax
from jax.experimental import pallas as pl
from jax.experimental.pallas import tpu as pltpu
from jax.experimental.pallas import tpu_sc as plsc

N = 10000
NP = 10240
E = 320000
D_IN = 128
D_H = 128
D_OUT = 64
DS = 128

NC = 2
NS = 16
CH = 128
CHUNK = 128
TOT = 2560
E_PAD = TOT * CHUNK
K0 = 80
K1 = 80

ROWS_PT = NP // NS
SUB = ROWS_PT // CH

BM = 1280
GM = NP // BM


@functools.lru_cache(maxsize=None)
def _make_segsum(ntab, with_deg):
  mesh = plsc.VectorSubcoreMesh(core_axis_name="c", subcore_axis_name="s",
                                num_cores=NC, num_subcores=NS)
  out_type = [jax.ShapeDtypeStruct((NC, NP, DS), jnp.float32)
              for _ in range(ntab)]
  if with_deg:
    out_type.append(jax.ShapeDtypeStruct((NC, NP), jnp.float32))
  scratch = [
      pltpu.VMEM((CHUNK,), jnp.int32),
      pltpu.VMEM((CHUNK,), jnp.int32),
      pltpu.VMEM((CHUNK,), jnp.int32),
      pltpu.VMEM((CHUNK,), jnp.int32),
      pltpu.VMEM((CHUNK,), jnp.int32),
      pltpu.VMEM((CHUNK,), jnp.int32),
      pltpu.VMEM((CHUNK,), jnp.int32),
      pltpu.VMEM((CHUNK,), jnp.int32),
      pltpu.VMEM((CHUNK, DS), jnp.float32),
      pltpu.VMEM((CHUNK, DS), jnp.float32),
      pltpu.VMEM((CHUNK,), jnp.float32),
      pltpu.VMEM((CH,), jnp.float32),
      pltpu.VMEM_SHARED((NP, DS), jnp.float32),
  ]
  if with_deg:
    scratch.append(pltpu.VMEM_SHARED((NP,), jnp.float32))
  scratch += [pltpu.SemaphoreType.DMA, pltpu.SemaphoreType.DMA,
              pltpu.SemaphoreType.DMA]

  def body(*args):
    tabs = args[:ntab]
    srcp, dstp = args[ntab], args[ntab + 1]
    rest = args[ntab + 2:]
    outs = rest[:ntab]
    rest = rest[ntab:]
    if with_deg:
      deg_out = rest[0]
      (s0, s1, s2, s3, d0, d1, d2, d3, buf_a, buf_b, ones, vec, acc, deg,
       sem_a, sem_b, sem_i) = rest[1:]
    else:
      (s0, s1, s2, s3, d0, d1, d2, d3, buf_a, buf_b, ones, vec, acc,
       sem_a, sem_b, sem_i) = rest
      deg = None
      deg_out = None
    sring = (s0, s1, s2, s3)
    dring = (d0, d1, d2, d3)

    c = lax.axis_index("c")
    s = lax.axis_index("s")
    base = s * ROWS_PT
    kb = jnp.where(c == 0, K0, K1)
    rowbase = jnp.where(c == 0, s * K0, NS * K0 + s * K1)

    zf = jnp.zeros((16,), jnp.float32)

    def zero_stage():
      def zrow(i, _):
        for jj in range(DS // 16):
          buf_a[i, pl.ds(jj * 16, 16)] = zf
        return 0
      lax.fori_loop(0, CH, zrow, 0)

    if with_deg:
      for jj in range(CH // 16):
        vec[pl.ds(jj * 16, 16)] = zf
      for b in range(SUB):
        pltpu.sync_copy(vec, deg.at[pl.ds(base + b * CH, CH)])
      of = jnp.ones((16,), jnp.float32)
      for jj in range(CHUNK // 16):
        ones[pl.ds(jj * 16, 16)] = of

    def row(j):
      return rowbase + jnp.minimum(j, kb - 1)

    def i_issue(j, b):
      r = row(j)
      pltpu.async_copy(srcp.at[r], sring[b], sem_i)
      pltpu.async_copy(dstp.at[r], dring[b], sem_i)

    def i_wait(j, b):
      r = row(j)
      pltpu.make_async_copy(srcp.at[r], sring[b], sem_i).wait()
      pltpu.make_async_copy(dstp.at[r], dring[b], sem_i).wait()

    for t in range(ntab):
      tbl = tabs[t]
      out = outs[t]
      deg_on = with_deg and t == 0

      def g_issue(b, buf, sem):
        pltpu.async_copy(tbl.at[sring[b]], buf, sem)

      def g_wait(b, buf, sem):
        pltpu.make_async_copy(tbl.at[sring[b]], buf, sem).wait()

      def scat(b, buf):
        pltpu.sync_copy(buf, acc.at[dring[b]], add=True)
        if deg_on:
          pltpu.sync_copy(ones, deg.at[dring[b]], add=True)

      zero_stage()
      for b in range(SUB):
        pltpu.sync_copy(buf_a.at[pl.ds(0, CH)],
                        acc.at[pl.ds(base + b * CH, CH)])
      plsc.subcore_barrier()

      i_issue(0, 0)
      i_issue(1, 1)
      i_wait(0, 0)
      g_issue(0, buf_a, sem_a)
      i_wait(1, 1)
      g_issue(1, buf_b, sem_b)
      i_issue(2, 2)
      i_issue(3, 3)

      def step(jj, _):
        j0 = 4 * jj
        for b in range(4):
          j = j0 + b
          buf, sem = (buf_a, sem_a) if b % 2 == 0 else (buf_b, sem_b)
          g_wait(b, buf, sem)
          scat(b, buf)
          i_wait(j + 2, (b + 2) % 4)
          g_issue((b + 2) % 4, buf, sem)
          i_issue(j + 4, b)
        return 0
      lax.fori_loop(0, kb // 4, step, 0)

      g_wait(0, buf_a, sem_a)
      g_wait(1, buf_b, sem_b)
      i_wait(0, 2)
      i_wait(0, 3)

      plsc.subcore_barrier()

      for b in range(SUB):
        off = base + b * CH
        pltpu.sync_copy(acc.at[pl.ds(off, CH)], buf_a.at[pl.ds(0, CH)])
        pltpu.sync_copy(buf_a.at[pl.ds(0, CH)], out.at[c, pl.ds(off, CH)])

    if with_deg:
      for b in range(SUB):
        off = base + b * CH
        pltpu.sync_copy(deg.at[pl.ds(off, CH)], vec)
        pltpu.sync_copy(vec, deg_out.at[c, pl.ds(off, CH)])

  return pl.kernel(body, out_type=tuple(out_type), mesh=mesh,
                   scratch_types=scratch,
                   compiler_params=pltpu.CompilerParams(
                       use_tc_tiling_on_sc=False))


def _tc1_body(x_ref, wl_ref, wr_ref, b_ref, p1_ref, pre1_ref):
  xb = x_ref[...]
  p1_ref[...] = jnp.dot(xb, wl_ref[...], preferred_element_type=jnp.float32)
  pre1_ref[...] = (jnp.dot(xb, wr_ref[...], preferred_element_type=jnp.float32)
                   + b_ref[...])


_tc1 = pl.pallas_call(
    _tc1_body,
    grid=(GM,),
    in_specs=[
        pl.BlockSpec((BM, D_IN), lambda i: (i, 0)),
        pl.BlockSpec((D_IN, D_H), lambda i: (0, 0)),
        pl.BlockSpec((D_IN, D_H), lambda i: (0, 0)),
        pl.BlockSpec((1, D_H), lambda i: (0, 0)),
    ],
    out_specs=[
        pl.BlockSpec((BM, D_H), lambda i: (i, 0)),
        pl.BlockSpec((BM, D_H), lambda i: (i, 0)),
    ],
    out_shape=[jax.ShapeDtypeStruct((NP, D_H), jnp.float32)] * 2,
)


def _tc2_body(s1a_ref, s1b_ref, dega_ref, degb_ref, pre1_ref, wl_ref,
              wr_ref, b2_ref, p2_ref, pre2_ref):
  deg = jnp.maximum(dega_ref[...] + degb_ref[...], 1.0)
  aggr = (s1a_ref[...] + s1b_ref[...]) / deg
  h = jnp.maximum(aggr + pre1_ref[...], 0.0)
  p2 = jnp.dot(h, wl_ref[...], preferred_element_type=jnp.float32)
  p2_ref[...] = jnp.concatenate([p2, jnp.zeros_like(p2)], axis=1)
  pre2_ref[...] = (jnp.dot(h, wr_ref[...], preferred_element_type=jnp.float32)
                   + b2_ref[...])


_tc2 = pl.pallas_call(
    _tc2_body,
    grid=(GM,),
    in_specs=[
        pl.BlockSpec((BM, D_H), lambda i: (i, 0)),
        pl.BlockSpec((BM, D_H), lambda i: (i, 0)),
        pl.BlockSpec((BM, 1), lambda i: (i, 0)),
        pl.BlockSpec((BM, 1), lambda i: (i, 0)),
        pl.BlockSpec((BM, D_H), lambda i: (i, 0)),
        pl.BlockSpec((D_H, D_OUT), lambda i: (0, 0)),
        pl.BlockSpec((D_H, D_OUT), lambda i: (0, 0)),
        pl.BlockSpec((1, D_OUT), lambda i: (0, 0)),
    ],
    out_specs=[
        pl.BlockSpec((BM, 2 * D_OUT), lambda i: (i, 0)),
        pl.BlockSpec((BM, D_OUT), lambda i: (i, 0)),
    ],
    out_shape=[
        jax.ShapeDtypeStruct((NP, 2 * D_OUT), jnp.float32),
        jax.ShapeDtypeStruct((NP, D_OUT), jnp.float32),
    ],
)


def _tc3_body(s2a_ref, s2b_ref, dega_ref, degb_ref, pre2_ref, y_ref, m_ref,
              out_ref, accs):
  i = pl.program_id(0)
  deg = jnp.maximum(dega_ref[...] + degb_ref[...], 1.0)
  s2 = (s2a_ref[...] + s2b_ref[...])[:, :D_OUT]
  z = s2 / deg + pre2_ref[...]
  zmax = jnp.max(z, axis=1, keepdims=True)
  lse = jnp.log(jnp.sum(jnp.exp(z - zmax), axis=1, keepdims=True)) + zmax
  logp = z - lse
  onehot = lax.broadcasted_iota(jnp.int32, z.shape, 1) == y_ref[...]
  picked = jnp.sum(jnp.where(onehot, logp, 0.0), axis=1)
  mv = m_ref[...][:, 0]
  pn = jnp.sum(picked * mv)
  pm = jnp.sum(mv)

  @pl.when(i == 0)
  def _():
    accs[0] = pn
    accs[1] = pm

  @pl.when(i > 0)
  def _():
    accs[0] += pn
    accs[1] += pm

  @pl.when(i == GM - 1)
  def _():
    out_ref[...] = jnp.full((1, 1), -accs[0] / jnp.maximum(accs[1], 1.0),
                            jnp.float32)


_tc3 = pl.pallas_call(
    _tc3_body,
    grid=(GM,),
    in_specs=[
        pl.BlockSpec((BM, 2 * D_OUT), lambda i: (i, 0)),
        pl.BlockSpec((BM, 2 * D_OUT), lambda i: (i, 0)),
        pl.BlockSpec((BM, 1), lambda i: (i, 0)),
        pl.BlockSpec((BM, 1), lambda i: (i, 0)),
        pl.BlockSpec((BM, D_OUT), lambda i: (i, 0)),
        pl.BlockSpec((BM, 1), lambda i: (i, 0)),
        pl.BlockSpec((BM, 1), lambda i: (i, 0)),
    ],
    out_specs=pl.BlockSpec((1, 1), lambda i: (0, 0)),
    out_shape=jax.ShapeDtypeStruct((1, 1), jnp.float32),
    scratch_shapes=[pltpu.SMEM((2,), jnp.float32)],
)


def kernel(x, edge_index, y, train_mask, Wl1, bl1, Wr1, br1, Wl2, bl2, Wr2,
           br2):
  src = edge_index[0]
  dst = edge_index[1]
  pad_e = E_PAD - E
  srcp = jnp.concatenate([src, jnp.zeros((pad_e,), jnp.int32)])
  srcp = srcp.reshape(TOT, CHUNK)
  dstp = jnp.concatenate([dst, jnp.full((pad_e,), N, jnp.int32)])
  dstp = dstp.reshape(TOT, CHUNK)

  xp = jnp.pad(x, ((0, NP - N), (0, 0)))
  b1 = (bl1 + br1).reshape(1, D_H)
  b2 = (bl2 + br2).reshape(1, D_OUT)

  p1, pre1 = _tc1(xp, Wl1, Wr1, b1)
  s1, degp = _make_segsum(1, True)(p1, srcp, dstp)
  dega = degp[0].reshape(NP, 1)
  degb = degp[1].reshape(NP, 1)
  p2p, pre2 = _tc2(s1[0], s1[1], dega, degb, pre1, Wl2, Wr2, b2)
  parts2, _deg2 = _make_segsum(1, True)(p2p, srcp, dstp)

  yp = jnp.pad(y, (0, NP - N)).reshape(NP, 1)
  mp = jnp.pad(train_mask.astype(jnp.float32), (0, NP - N)).reshape(NP, 1)
  loss = _tc3(parts2[0], parts2[1], dega, degb, pre2, yp, mp)
  return loss.reshape(1)


# speedup vs baseline: 1.2044x; 1.2044x over previous
import functools

import jax
import jax.numpy as jnp
from jax import lax
from jax.experimental import pallas as pl
from jax.experimental.pallas import tpu as pltpu
from jax.experimental.pallas import tpu_sc as plsc

N = 10000
NP = 10240
E = 320000
D_IN = 128
D_H = 128
D_OUT = 64
DS = 64

NC = 2
NS = 16
CH = 128
CHUNK = 320
TOT = 1024
E_PAD = TOT * CHUNK
K0 = 32
K1 = 32

ROWS_PT = NP // NS
SUB = ROWS_PT // CH

BM = 1280
GM = NP // BM


@functools.lru_cache(maxsize=None)
def _make_segsum(ntab, with_deg):
  mesh = plsc.VectorSubcoreMesh(core_axis_name="c", subcore_axis_name="s",
                                num_cores=NC, num_subcores=NS)
  out_type = [jax.ShapeDtypeStruct((NC, NP, DS), jnp.float32)
              for _ in range(ntab)]
  if with_deg:
    out_type.append(jax.ShapeDtypeStruct((NC, NP), jnp.float32))
  scratch = [
      pltpu.VMEM((CHUNK,), jnp.int32),
      pltpu.VMEM((CHUNK,), jnp.int32),
      pltpu.VMEM((CHUNK,), jnp.int32),
      pltpu.VMEM((CHUNK,), jnp.int32),
      pltpu.VMEM((CHUNK,), jnp.int32),
      pltpu.VMEM((CHUNK,), jnp.int32),
      pltpu.VMEM((CHUNK,), jnp.int32),
      pltpu.VMEM((CHUNK,), jnp.int32),
      pltpu.VMEM((CHUNK, DS), jnp.float32),
      pltpu.VMEM((CHUNK, DS), jnp.float32),
      pltpu.VMEM((CHUNK,), jnp.float32),
      pltpu.VMEM((CH,), jnp.float32),
      pltpu.VMEM_SHARED((NP, DS), jnp.float32),
  ]
  if with_deg:
    scratch.append(pltpu.VMEM_SHARED((NP,), jnp.float32))
  scratch += [pltpu.SemaphoreType.DMA, pltpu.SemaphoreType.DMA,
              pltpu.SemaphoreType.DMA]

  def body(*args):
    tabs = args[:ntab]
    srcp, dstp = args[ntab], args[ntab + 1]
    rest = args[ntab + 2:]
    outs = rest[:ntab]
    rest = rest[ntab:]
    if with_deg:
      deg_out = rest[0]
      (s0, s1, s2, s3, d0, d1, d2, d3, buf_a, buf_b, ones, vec, acc, deg,
       sem_a, sem_b, sem_i) = rest[1:]
    else:
      (s0, s1, s2, s3, d0, d1, d2, d3, buf_a, buf_b, ones, vec, acc,
       sem_a, sem_b, sem_i) = rest
      deg = None
      deg_out = None
    sring = (s0, s1, s2, s3)
    dring = (d0, d1, d2, d3)

    c = lax.axis_index("c")
    s = lax.axis_index("s")
    base = s * ROWS_PT
    kb = jnp.where(c == 0, K0, K1)
    rowbase = jnp.where(c == 0, s * K0, NS * K0 + s * K1)

    zf = jnp.zeros((16,), jnp.float32)

    def zero_stage():
      def zrow(i, _):
        for jj in range(DS // 16):
          buf_a[i, pl.ds(jj * 16, 16)] = zf
        return 0
      lax.fori_loop(0, CH, zrow, 0)

    if with_deg:
      for jj in range(CH // 16):
        vec[pl.ds(jj * 16, 16)] = zf
      for b in range(SUB):
        pltpu.sync_copy(vec, deg.at[pl.ds(base + b * CH, CH)])
      of = jnp.ones((16,), jnp.float32)
      for jj in range(CHUNK // 16):
        ones[pl.ds(jj * 16, 16)] = of

    def row(j):
      return rowbase + jnp.minimum(j, kb - 1)

    def i_issue(j, b):
      r = row(j)
      pltpu.async_copy(srcp.at[r], sring[b], sem_i)
      pltpu.async_copy(dstp.at[r], dring[b], sem_i)

    def i_wait(j, b):
      r = row(j)
      pltpu.make_async_copy(srcp.at[r], sring[b], sem_i).wait()
      pltpu.make_async_copy(dstp.at[r], dring[b], sem_i).wait()

    for t in range(ntab):
      tbl = tabs[t]
      out = outs[t]
      deg_on = with_deg and t == 0

      def g_issue(b, buf, sem):
        pltpu.async_copy(tbl.at[sring[b]], buf, sem)

      def g_wait(b, buf, sem):
        pltpu.make_async_copy(tbl.at[sring[b]], buf, sem).wait()

      def scat(b, buf):
        pltpu.sync_copy(buf, acc.at[dring[b]], add=True)
        if deg_on:
          pltpu.sync_copy(ones, deg.at[dring[b]], add=True)

      zero_stage()
      for b in range(SUB):
        pltpu.sync_copy(buf_a.at[pl.ds(0, CH)],
                        acc.at[pl.ds(base + b * CH, CH)])
      plsc.subcore_barrier()

      i_issue(0, 0)
      i_issue(1, 1)
      i_wait(0, 0)
      g_issue(0, buf_a, sem_a)
      i_wait(1, 1)
      g_issue(1, buf_b, sem_b)
      i_issue(2, 2)
      i_issue(3, 3)

      def step(jj, _):
        j0 = 4 * jj
        for b in range(4):
          j = j0 + b
          buf, sem = (buf_a, sem_a) if b % 2 == 0 else (buf_b, sem_b)
          g_wait(b, buf, sem)
          scat(b, buf)
          i_wait(j + 2, (b + 2) % 4)
          g_issue((b + 2) % 4, buf, sem)
          i_issue(j + 4, b)
        return 0
      lax.fori_loop(0, kb // 4, step, 0)

      g_wait(0, buf_a, sem_a)
      g_wait(1, buf_b, sem_b)
      i_wait(0, 2)
      i_wait(0, 3)

      plsc.subcore_barrier()

      for b in range(SUB):
        off = base + b * CH
        pltpu.sync_copy(acc.at[pl.ds(off, CH)], buf_a.at[pl.ds(0, CH)])
        pltpu.sync_copy(buf_a.at[pl.ds(0, CH)], out.at[c, pl.ds(off, CH)])

    if with_deg:
      for b in range(SUB):
        off = base + b * CH
        pltpu.sync_copy(deg.at[pl.ds(off, CH)], vec)
        pltpu.sync_copy(vec, deg_out.at[c, pl.ds(off, CH)])

  return pl.kernel(body, out_type=tuple(out_type), mesh=mesh,
                   scratch_types=scratch,
                   compiler_params=pltpu.CompilerParams(
                       use_tc_tiling_on_sc=False))


def _tc1_body(x_ref, wla_ref, wlb_ref, wr_ref, b_ref, p1a_ref, p1b_ref,
              pre1_ref):
  xb = x_ref[...]
  p1a_ref[...] = jnp.dot(xb, wla_ref[...], preferred_element_type=jnp.float32)
  p1b_ref[...] = jnp.dot(xb, wlb_ref[...], preferred_element_type=jnp.float32)
  pre1_ref[...] = (jnp.dot(xb, wr_ref[...], preferred_element_type=jnp.float32)
                   + b_ref[...])


_tc1 = pl.pallas_call(
    _tc1_body,
    grid=(GM,),
    in_specs=[
        pl.BlockSpec((BM, D_IN), lambda i: (i, 0)),
        pl.BlockSpec((D_IN, DS), lambda i: (0, 0)),
        pl.BlockSpec((D_IN, DS), lambda i: (0, 0)),
        pl.BlockSpec((D_IN, D_H), lambda i: (0, 0)),
        pl.BlockSpec((1, D_H), lambda i: (0, 0)),
    ],
    out_specs=[
        pl.BlockSpec((BM, DS), lambda i: (i, 0)),
        pl.BlockSpec((BM, DS), lambda i: (i, 0)),
        pl.BlockSpec((BM, D_H), lambda i: (i, 0)),
    ],
    out_shape=[
        jax.ShapeDtypeStruct((NP, DS), jnp.float32),
        jax.ShapeDtypeStruct((NP, DS), jnp.float32),
        jax.ShapeDtypeStruct((NP, D_H), jnp.float32),
    ],
)


def _tc2_body(sa0_ref, sa1_ref, sb0_ref, sb1_ref, dega_ref, degb_ref,
              pre1_ref, wl_ref, wr_ref, b2_ref, p2_ref, pre2_ref):
  deg = jnp.maximum(dega_ref[...] + degb_ref[...], 1.0)
  inv = 1.0 / deg
  aggr_lo = (sa0_ref[...] + sa1_ref[...]) * inv
  aggr_hi = (sb0_ref[...] + sb1_ref[...]) * inv
  pre1 = pre1_ref[...]
  h_lo = jnp.maximum(aggr_lo + pre1[:, :DS], 0.0)
  h_hi = jnp.maximum(aggr_hi + pre1[:, DS:], 0.0)
  h = jnp.concatenate([h_lo, h_hi], axis=1)
  p2_ref[...] = jnp.dot(h, wl_ref[...], preferred_element_type=jnp.float32)
  pre2_ref[...] = (jnp.dot(h, wr_ref[...], preferred_element_type=jnp.float32)
                   + b2_ref[...])


_tc2 = pl.pallas_call(
    _tc2_body,
    grid=(GM,),
    in_specs=[
        pl.BlockSpec((BM, DS), lambda i: (i, 0)),
        pl.BlockSpec((BM, DS), lambda i: (i, 0)),
        pl.BlockSpec((BM, DS), lambda i: (i, 0)),
        pl.BlockSpec((BM, DS), lambda i: (i, 0)),
        pl.BlockSpec((BM, 1), lambda i: (i, 0)),
        pl.BlockSpec((BM, 1), lambda i: (i, 0)),
        pl.BlockSpec((BM, D_H), lambda i: (i, 0)),
        pl.BlockSpec((D_H, D_OUT), lambda i: (0, 0)),
        pl.BlockSpec((D_H, D_OUT), lambda i: (0, 0)),
        pl.BlockSpec((1, D_OUT), lambda i: (0, 0)),
    ],
    out_specs=[
        pl.BlockSpec((BM, D_OUT), lambda i: (i, 0)),
        pl.BlockSpec((BM, D_OUT), lambda i: (i, 0)),
    ],
    out_shape=[jax.ShapeDtypeStruct((NP, D_OUT), jnp.float32)] * 2,
)


def _tc3_body(s2a_ref, s2b_ref, dega_ref, degb_ref, pre2_ref, y_ref, m_ref,
              out_ref, accs):
  i = pl.program_id(0)
  deg = jnp.maximum(dega_ref[...] + degb_ref[...], 1.0)
  z = (s2a_ref[...] + s2b_ref[...]) / deg + pre2_ref[...]
  zmax = jnp.max(z, axis=1, keepdims=True)
  lse = jnp.log(jnp.sum(jnp.exp(z - zmax), axis=1, keepdims=True)) + zmax
  logp = z - lse
  onehot = lax.broadcasted_iota(jnp.int32, z.shape, 1) == y_ref[...]
  picked = jnp.sum(jnp.where(onehot, logp, 0.0), axis=1)
  mv = m_ref[...][:, 0]
  pn = jnp.sum(picked * mv)
  pm = jnp.sum(mv)

  @pl.when(i == 0)
  def _():
    accs[0] = pn
    accs[1] = pm

  @pl.when(i > 0)
  def _():
    accs[0] += pn
    accs[1] += pm

  @pl.when(i == GM - 1)
  def _():
    out_ref[...] = jnp.full((1, 1), -accs[0] / jnp.maximum(accs[1], 1.0),
                            jnp.float32)


_tc3 = pl.pallas_call(
    _tc3_body,
    grid=(GM,),
    in_specs=[
        pl.BlockSpec((BM, D_OUT), lambda i: (i, 0)),
        pl.BlockSpec((BM, D_OUT), lambda i: (i, 0)),
        pl.BlockSpec((BM, 1), lambda i: (i, 0)),
        pl.BlockSpec((BM, 1), lambda i: (i, 0)),
        pl.BlockSpec((BM, D_OUT), lambda i: (i, 0)),
        pl.BlockSpec((BM, 1), lambda i: (i, 0)),
        pl.BlockSpec((BM, 1), lambda i: (i, 0)),
    ],
    out_specs=pl.BlockSpec((1, 1), lambda i: (0, 0)),
    out_shape=jax.ShapeDtypeStruct((1, 1), jnp.float32),
    scratch_shapes=[pltpu.SMEM((2,), jnp.float32)],
)


def kernel(x, edge_index, y, train_mask, Wl1, bl1, Wr1, br1, Wl2, bl2, Wr2,
           br2):
  src = edge_index[0]
  dst = edge_index[1]
  pad_e = E_PAD - E
  srcp = jnp.concatenate([src, jnp.zeros((pad_e,), jnp.int32)])
  srcp = srcp.reshape(TOT, CHUNK)
  dstp = jnp.concatenate([dst, jnp.full((pad_e,), N, jnp.int32)])
  dstp = dstp.reshape(TOT, CHUNK)

  xp = jnp.pad(x, ((0, NP - N), (0, 0)))
  b1 = (bl1 + br1).reshape(1, D_H)
  b2 = (bl2 + br2).reshape(1, D_OUT)

  p1a, p1b, pre1 = _tc1(xp, Wl1[:, :DS], Wl1[:, DS:], Wr1, b1)
  sa, sb, degp = _make_segsum(2, True)(p1a, p1b, srcp, dstp)
  dega = degp[0].reshape(NP, 1)
  degb = degp[1].reshape(NP, 1)
  p2, pre2 = _tc2(sa[0], sa[1], sb[0], sb[1], dega, degb, pre1, Wl2, Wr2, b2)
  (parts2,) = _make_segsum(1, False)(p2, srcp, dstp)

  yp = jnp.pad(y, (0, NP - N)).reshape(NP, 1)
  mp = jnp.pad(train_mask.astype(jnp.float32), (0, NP - N)).reshape(NP, 1)
  loss = _tc3(parts2[0], parts2[1], dega, degb, pre2, yp, mp)
  return loss.reshape(1)


# speedup vs baseline: 1.2195x; 1.0126x over previous
import functools

import jax
import jax.numpy as jnp
from jax import lax
from jax.experimental import pallas as pl
from jax.experimental.pallas import tpu as pltpu
from jax.experimental.pallas import tpu_sc as plsc

N = 10000
NP = 10240
E = 320000
D_IN = 128
D_H = 128
D_OUT = 64
DS = 64

NC = 2
NS = 16
CH = 128
CHUNK = 256
TOT = 1280
E_PAD = TOT * CHUNK
K0 = 40
K1 = 40

ROWS_PT = NP // NS
SUB = ROWS_PT // CH

BM = 1280
GM = NP // BM


@functools.lru_cache(maxsize=None)
def _make_segsum(ntab, with_deg):
  mesh = plsc.VectorSubcoreMesh(core_axis_name="c", subcore_axis_name="s",
                                num_cores=NC, num_subcores=NS)
  out_type = [jax.ShapeDtypeStruct((NC, NP, DS), jnp.float32)
              for _ in range(ntab)]
  if with_deg:
    out_type.append(jax.ShapeDtypeStruct((NC, NP), jnp.float32))
  scratch = [
      pltpu.VMEM((CHUNK,), jnp.int32),
      pltpu.VMEM((CHUNK,), jnp.int32),
      pltpu.VMEM((CHUNK,), jnp.int32),
      pltpu.VMEM((CHUNK,), jnp.int32),
      pltpu.VMEM((CHUNK,), jnp.int32),
      pltpu.VMEM((CHUNK,), jnp.int32),
      pltpu.VMEM((CHUNK,), jnp.int32),
      pltpu.VMEM((CHUNK,), jnp.int32),
      pltpu.VMEM((CHUNK, DS), jnp.float32),
      pltpu.VMEM((CHUNK, DS), jnp.float32),
      pltpu.VMEM((CHUNK,), jnp.float32),
      pltpu.VMEM((CH,), jnp.float32),
      pltpu.VMEM_SHARED((NP, DS), jnp.float32),
  ]
  if with_deg:
    scratch.append(pltpu.VMEM_SHARED((NP,), jnp.float32))
  scratch += [pltpu.SemaphoreType.DMA, pltpu.SemaphoreType.DMA,
              pltpu.SemaphoreType.DMA]

  def body(*args):
    tabs = args[:ntab]
    srcp, dstp = args[ntab], args[ntab + 1]
    rest = args[ntab + 2:]
    outs = rest[:ntab]
    rest = rest[ntab:]
    if with_deg:
      deg_out = rest[0]
      (s0, s1, s2, s3, d0, d1, d2, d3, buf_a, buf_b, ones, vec, acc, deg,
       sem_a, sem_b, sem_i) = rest[1:]
    else:
      (s0, s1, s2, s3, d0, d1, d2, d3, buf_a, buf_b, ones, vec, acc,
       sem_a, sem_b, sem_i) = rest
      deg = None
      deg_out = None
    sring = (s0, s1, s2, s3)
    dring = (d0, d1, d2, d3)

    c = lax.axis_index("c")
    s = lax.axis_index("s")
    base = s * ROWS_PT
    kb = jnp.where(c == 0, K0, K1)
    rowbase = jnp.where(c == 0, s * K0, NS * K0 + s * K1)

    zf = jnp.zeros((16,), jnp.float32)

    def zero_stage():
      def zrow(i, _):
        for jj in range(DS // 16):
          buf_a[i, pl.ds(jj * 16, 16)] = zf
        return 0
      lax.fori_loop(0, CH, zrow, 0)

    if with_deg:
      for jj in range(CH // 16):
        vec[pl.ds(jj * 16, 16)] = zf
      for b in range(SUB):
        pltpu.sync_copy(vec, deg.at[pl.ds(base + b * CH, CH)])
      of = jnp.ones((16,), jnp.float32)
      for jj in range(CHUNK // 16):
        ones[pl.ds(jj * 16, 16)] = of

    def row(j):
      return rowbase + jnp.minimum(j, kb - 1)

    def i_issue(j, b):
      r = row(j)
      pltpu.async_copy(srcp.at[r], sring[b], sem_i)
      pltpu.async_copy(dstp.at[r], dring[b], sem_i)

    def i_wait(j, b):
      r = row(j)
      pltpu.make_async_copy(srcp.at[r], sring[b], sem_i).wait()
      pltpu.make_async_copy(dstp.at[r], dring[b], sem_i).wait()

    for t in range(ntab):
      tbl = tabs[t]
      out = outs[t]
      deg_on = with_deg and t == 0

      def g_issue(b, buf, sem):
        pltpu.async_copy(tbl.at[sring[b]], buf, sem)

      def g_wait(b, buf, sem):
        pltpu.make_async_copy(tbl.at[sring[b]], buf, sem).wait()

      def scat(b, buf):
        pltpu.sync_copy(buf, acc.at[dring[b]], add=True)
        if deg_on:
          pltpu.sync_copy(ones, deg.at[dring[b]], add=True)

      zero_stage()
      for b in range(SUB):
        pltpu.sync_copy(buf_a.at[pl.ds(0, CH)],
                        acc.at[pl.ds(base + b * CH, CH)])
      plsc.subcore_barrier()

      i_issue(0, 0)
      i_issue(1, 1)
      i_wait(0, 0)
      g_issue(0, buf_a, sem_a)
      i_wait(1, 1)
      g_issue(1, buf_b, sem_b)
      i_issue(2, 2)
      i_issue(3, 3)

      def step(jj, _):
        j0 = 4 * jj
        for b in range(4):
          j = j0 + b
          buf, sem = (buf_a, sem_a) if b % 2 == 0 else (buf_b, sem_b)
          g_wait(b, buf, sem)
          scat(b, buf)
          i_wait(j + 2, (b + 2) % 4)
          g_issue((b + 2) % 4, buf, sem)
          i_issue(j + 4, b)
        return 0
      lax.fori_loop(0, kb // 4, step, 0)

      g_wait(0, buf_a, sem_a)
      g_wait(1, buf_b, sem_b)
      i_wait(0, 2)
      i_wait(0, 3)

      plsc.subcore_barrier()

      for b in range(SUB):
        off = base + b * CH
        pltpu.sync_copy(acc.at[pl.ds(off, CH)], buf_a.at[pl.ds(0, CH)])
        pltpu.sync_copy(buf_a.at[pl.ds(0, CH)], out.at[c, pl.ds(off, CH)])

    if with_deg:
      for b in range(SUB):
        off = base + b * CH
        pltpu.sync_copy(deg.at[pl.ds(off, CH)], vec)
        pltpu.sync_copy(vec, deg_out.at[c, pl.ds(off, CH)])

  return pl.kernel(body, out_type=tuple(out_type), mesh=mesh,
                   scratch_types=scratch,
                   compiler_params=pltpu.CompilerParams(
                       use_tc_tiling_on_sc=False))


def _tc1_body(x_ref, wla_ref, wlb_ref, wr_ref, b_ref, p1a_ref, p1b_ref,
              pre1_ref):
  xb = x_ref[...]
  p1a_ref[...] = jnp.dot(xb, wla_ref[...], preferred_element_type=jnp.float32)
  p1b_ref[...] = jnp.dot(xb, wlb_ref[...], preferred_element_type=jnp.float32)
  pre1_ref[...] = (jnp.dot(xb, wr_ref[...], preferred_element_type=jnp.float32)
                   + b_ref[...])


_tc1 = pl.pallas_call(
    _tc1_body,
    grid=(GM,),
    in_specs=[
        pl.BlockSpec((BM, D_IN), lambda i: (i, 0)),
        pl.BlockSpec((D_IN, DS), lambda i: (0, 0)),
        pl.BlockSpec((D_IN, DS), lambda i: (0, 0)),
        pl.BlockSpec((D_IN, D_H), lambda i: (0, 0)),
        pl.BlockSpec((1, D_H), lambda i: (0, 0)),
    ],
    out_specs=[
        pl.BlockSpec((BM, DS), lambda i: (i, 0)),
        pl.BlockSpec((BM, DS), lambda i: (i, 0)),
        pl.BlockSpec((BM, D_H), lambda i: (i, 0)),
    ],
    out_shape=[
        jax.ShapeDtypeStruct((NP, DS), jnp.float32),
        jax.ShapeDtypeStruct((NP, DS), jnp.float32),
        jax.ShapeDtypeStruct((NP, D_H), jnp.float32),
    ],
)


def _tc2_body(sa0_ref, sa1_ref, sb0_ref, sb1_ref, dega_ref, degb_ref,
              pre1_ref, wl_ref, wr_ref, b2_ref, p2_ref, pre2_ref):
  deg = jnp.maximum(dega_ref[...] + degb_ref[...], 1.0)
  inv = 1.0 / deg
  aggr_lo = (sa0_ref[...] + sa1_ref[...]) * inv
  aggr_hi = (sb0_ref[...] + sb1_ref[...]) * inv
  pre1 = pre1_ref[...]
  h_lo = jnp.maximum(aggr_lo + pre1[:, :DS], 0.0)
  h_hi = jnp.maximum(aggr_hi + pre1[:, DS:], 0.0)
  h = jnp.concatenate([h_lo, h_hi], axis=1)
  p2_ref[...] = jnp.dot(h, wl_ref[...], preferred_element_type=jnp.float32)
  pre2_ref[...] = (jnp.dot(h, wr_ref[...], preferred_element_type=jnp.float32)
                   + b2_ref[...])


_tc2 = pl.pallas_call(
    _tc2_body,
    grid=(GM,),
    in_specs=[
        pl.BlockSpec((BM, DS), lambda i: (i, 0)),
        pl.BlockSpec((BM, DS), lambda i: (i, 0)),
        pl.BlockSpec((BM, DS), lambda i: (i, 0)),
        pl.BlockSpec((BM, DS), lambda i: (i, 0)),
        pl.BlockSpec((BM, 1), lambda i: (i, 0)),
        pl.BlockSpec((BM, 1), lambda i: (i, 0)),
        pl.BlockSpec((BM, D_H), lambda i: (i, 0)),
        pl.BlockSpec((D_H, D_OUT), lambda i: (0, 0)),
        pl.BlockSpec((D_H, D_OUT), lambda i: (0, 0)),
        pl.BlockSpec((1, D_OUT), lambda i: (0, 0)),
    ],
    out_specs=[
        pl.BlockSpec((BM, D_OUT), lambda i: (i, 0)),
        pl.BlockSpec((BM, D_OUT), lambda i: (i, 0)),
    ],
    out_shape=[jax.ShapeDtypeStruct((NP, D_OUT), jnp.float32)] * 2,
)


def _tc3_body(s2a_ref, s2b_ref, dega_ref, degb_ref, pre2_ref, y_ref, m_ref,
              out_ref, accs):
  i = pl.program_id(0)
  deg = jnp.maximum(dega_ref[...] + degb_ref[...], 1.0)
  z = (s2a_ref[...] + s2b_ref[...]) / deg + pre2_ref[...]
  zmax = jnp.max(z, axis=1, keepdims=True)
  lse = jnp.log(jnp.sum(jnp.exp(z - zmax), axis=1, keepdims=True)) + zmax
  logp = z - lse
  onehot = lax.broadcasted_iota(jnp.int32, z.shape, 1) == y_ref[...]
  picked = jnp.sum(jnp.where(onehot, logp, 0.0), axis=1)
  mv = m_ref[...][:, 0]
  pn = jnp.sum(picked * mv)
  pm = jnp.sum(mv)

  @pl.when(i == 0)
  def _():
    accs[0] = pn
    accs[1] = pm

  @pl.when(i > 0)
  def _():
    accs[0] += pn
    accs[1] += pm

  @pl.when(i == GM - 1)
  def _():
    out_ref[...] = jnp.full((1, 1), -accs[0] / jnp.maximum(accs[1], 1.0),
                            jnp.float32)


_tc3 = pl.pallas_call(
    _tc3_body,
    grid=(GM,),
    in_specs=[
        pl.BlockSpec((BM, D_OUT), lambda i: (i, 0)),
        pl.BlockSpec((BM, D_OUT), lambda i: (i, 0)),
        pl.BlockSpec((BM, 1), lambda i: (i, 0)),
        pl.BlockSpec((BM, 1), lambda i: (i, 0)),
        pl.BlockSpec((BM, D_OUT), lambda i: (i, 0)),
        pl.BlockSpec((BM, 1), lambda i: (i, 0)),
        pl.BlockSpec((BM, 1), lambda i: (i, 0)),
    ],
    out_specs=pl.BlockSpec((1, 1), lambda i: (0, 0)),
    out_shape=jax.ShapeDtypeStruct((1, 1), jnp.float32),
    scratch_shapes=[pltpu.SMEM((2,), jnp.float32)],
)


def kernel(x, edge_index, y, train_mask, Wl1, bl1, Wr1, br1, Wl2, bl2, Wr2,
           br2):
  src = edge_index[0]
  dst = edge_index[1]
  pad_e = E_PAD - E
  srcp = jnp.concatenate([src, jnp.zeros((pad_e,), jnp.int32)])
  srcp = srcp.reshape(TOT, CHUNK)
  dstp = jnp.concatenate([dst, jnp.full((pad_e,), N, jnp.int32)])
  dstp = dstp.reshape(TOT, CHUNK)

  xp = jnp.pad(x, ((0, NP - N), (0, 0)))
  b1 = (bl1 + br1).reshape(1, D_H)
  b2 = (bl2 + br2).reshape(1, D_OUT)

  p1a, p1b, pre1 = _tc1(xp, Wl1[:, :DS], Wl1[:, DS:], Wr1, b1)
  sa, sb, degp = _make_segsum(2, True)(p1a, p1b, srcp, dstp)
  dega = degp[0].reshape(NP, 1)
  degb = degp[1].reshape(NP, 1)
  p2, pre2 = _tc2(sa[0], sa[1], sb[0], sb[1], dega, degb, pre1, Wl2, Wr2, b2)
  (parts2,) = _make_segsum(1, False)(p2, srcp, dstp)

  yp = jnp.pad(y, (0, NP - N)).reshape(NP, 1)
  mp = jnp.pad(train_mask.astype(jnp.float32), (0, NP - N)).reshape(NP, 1)
  loss = _tc3(parts2[0], parts2[1], dega, degb, pre2, yp, mp)
  return loss.reshape(1)


# speedup vs baseline: 2.4364x; 1.9978x over previous
import functools

import jax
import jax.numpy as jnp
from jax import lax
from jax.experimental import pallas as pl
from jax.experimental.pallas import tpu as pltpu
from jax.experimental.pallas import tpu_sc as plsc

N = 10000
NP = 10240
E = 320000
D_IN = 128
D_H = 128
D_OUT = 64
DS = 64

NC = 2
NS = 16
CH = 128
CHUNK = 256
TOT = 1280
E_PAD = TOT * CHUNK
K0 = 40
K1 = 40

ROWS_PT = NP // NS
SUB = ROWS_PT // CH

BM = 1280
GM = NP // BM


@functools.lru_cache(maxsize=None)
def _make_segsum(ntab, with_deg):
  mesh = plsc.VectorSubcoreMesh(core_axis_name="c", subcore_axis_name="s",
                                num_cores=NC, num_subcores=NS)
  out_type = [jax.ShapeDtypeStruct((NC, NP, DS), jnp.float32)
              for _ in range(ntab)]
  if with_deg:
    out_type.append(jax.ShapeDtypeStruct((NC, NP), jnp.float32))
  scratch = [
      pltpu.VMEM((CHUNK,), jnp.int32),
      pltpu.VMEM((CHUNK,), jnp.int32),
      pltpu.VMEM((CHUNK,), jnp.int32),
      pltpu.VMEM((CHUNK,), jnp.int32),
      pltpu.VMEM((CHUNK,), jnp.int32),
      pltpu.VMEM((CHUNK,), jnp.int32),
      pltpu.VMEM((CHUNK,), jnp.int32),
      pltpu.VMEM((CHUNK,), jnp.int32),
      pltpu.VMEM((CHUNK, DS), jnp.float32),
      pltpu.VMEM((CHUNK, DS), jnp.float32),
      pltpu.VMEM((CHUNK,), jnp.float32),
      pltpu.VMEM((CH,), jnp.float32),
      pltpu.VMEM_SHARED((NP, DS), jnp.float32),
      pltpu.VMEM_SHARED((NP, DS), jnp.float32),
  ]
  if with_deg:
    scratch.append(pltpu.VMEM_SHARED((NP,), jnp.float32))
  scratch += [pltpu.SemaphoreType.DMA, pltpu.SemaphoreType.DMA,
              pltpu.SemaphoreType.DMA]

  def body(*args):
    tabs = args[:ntab]
    srcp, dstp = args[ntab], args[ntab + 1]
    rest = args[ntab + 2:]
    outs = rest[:ntab]
    rest = rest[ntab:]
    if with_deg:
      deg_out = rest[0]
      (s0, s1, s2, s3, d0, d1, d2, d3, buf_a, buf_b, ones, vec, acc, tbl_sh,
       deg, sem_a, sem_b, sem_i) = rest[1:]
    else:
      (s0, s1, s2, s3, d0, d1, d2, d3, buf_a, buf_b, ones, vec, acc, tbl_sh,
       sem_a, sem_b, sem_i) = rest
      deg = None
      deg_out = None
    sring = (s0, s1, s2, s3)
    dring = (d0, d1, d2, d3)

    c = lax.axis_index("c")
    s = lax.axis_index("s")
    base = s * ROWS_PT
    kb = jnp.where(c == 0, K0, K1)
    rowbase = jnp.where(c == 0, s * K0, NS * K0 + s * K1)

    zf = jnp.zeros((16,), jnp.float32)

    def zero_stage():
      def zrow(i, _):
        for jj in range(DS // 16):
          buf_a[i, pl.ds(jj * 16, 16)] = zf
        return 0
      lax.fori_loop(0, CH, zrow, 0)

    if with_deg:
      for jj in range(CH // 16):
        vec[pl.ds(jj * 16, 16)] = zf
      for b in range(SUB):
        pltpu.sync_copy(vec, deg.at[pl.ds(base + b * CH, CH)])
      of = jnp.ones((16,), jnp.float32)
      for jj in range(CHUNK // 16):
        ones[pl.ds(jj * 16, 16)] = of

    def row(j):
      return rowbase + jnp.minimum(j, kb - 1)

    def i_issue(j, b):
      r = row(j)
      pltpu.async_copy(srcp.at[r], sring[b], sem_i)
      pltpu.async_copy(dstp.at[r], dring[b], sem_i)

    def i_wait(j, b):
      r = row(j)
      pltpu.make_async_copy(srcp.at[r], sring[b], sem_i).wait()
      pltpu.make_async_copy(dstp.at[r], dring[b], sem_i).wait()

    for t in range(ntab):
      tbl = tabs[t]
      out = outs[t]
      deg_on = with_deg and t == 0

      for bb in range(ROWS_PT // CH):
        off = base + bb * CH
        pltpu.sync_copy(tbl.at[pl.ds(off, CH)], buf_a.at[pl.ds(0, CH)])
        pltpu.sync_copy(buf_a.at[pl.ds(0, CH)], tbl_sh.at[pl.ds(off, CH)])

      def g_issue(b, buf, sem):
        pltpu.async_copy(tbl_sh.at[sring[b]], buf, sem)

      def g_wait(b, buf, sem):
        pltpu.make_async_copy(tbl_sh.at[sring[b]], buf, sem).wait()

      def scat(b, buf):
        pltpu.sync_copy(buf, acc.at[dring[b]], add=True)
        if deg_on:
          pltpu.sync_copy(ones, deg.at[dring[b]], add=True)

      zero_stage()
      for b in range(SUB):
        pltpu.sync_copy(buf_a.at[pl.ds(0, CH)],
                        acc.at[pl.ds(base + b * CH, CH)])
      plsc.subcore_barrier()

      i_issue(0, 0)
      i_issue(1, 1)
      i_wait(0, 0)
      g_issue(0, buf_a, sem_a)
      i_wait(1, 1)
      g_issue(1, buf_b, sem_b)
      i_issue(2, 2)
      i_issue(3, 3)

      def step(jj, _):
        j0 = 4 * jj
        for b in range(4):
          j = j0 + b
          buf, sem = (buf_a, sem_a) if b % 2 == 0 else (buf_b, sem_b)
          g_wait(b, buf, sem)
          scat(b, buf)
          i_wait(j + 2, (b + 2) % 4)
          g_issue((b + 2) % 4, buf, sem)
          i_issue(j + 4, b)
        return 0
      lax.fori_loop(0, kb // 4, step, 0)

      g_wait(0, buf_a, sem_a)
      g_wait(1, buf_b, sem_b)
      i_wait(0, 2)
      i_wait(0, 3)

      plsc.subcore_barrier()

      for b in range(SUB):
        off = base + b * CH
        pltpu.sync_copy(acc.at[pl.ds(off, CH)], buf_a.at[pl.ds(0, CH)])
        pltpu.sync_copy(buf_a.at[pl.ds(0, CH)], out.at[c, pl.ds(off, CH)])

    if with_deg:
      for b in range(SUB):
        off = base + b * CH
        pltpu.sync_copy(deg.at[pl.ds(off, CH)], vec)
        pltpu.sync_copy(vec, deg_out.at[c, pl.ds(off, CH)])

  return pl.kernel(body, out_type=tuple(out_type), mesh=mesh,
                   scratch_types=scratch,
                   compiler_params=pltpu.CompilerParams(
                       use_tc_tiling_on_sc=False))


def _tc1_body(x_ref, wla_ref, wlb_ref, wr_ref, b_ref, p1a_ref, p1b_ref,
              pre1_ref):
  xb = x_ref[...]
  p1a_ref[...] = jnp.dot(xb, wla_ref[...], preferred_element_type=jnp.float32)
  p1b_ref[...] = jnp.dot(xb, wlb_ref[...], preferred_element_type=jnp.float32)
  pre1_ref[...] = (jnp.dot(xb, wr_ref[...], preferred_element_type=jnp.float32)
                   + b_ref[...])


_tc1 = pl.pallas_call(
    _tc1_body,
    grid=(GM,),
    in_specs=[
        pl.BlockSpec((BM, D_IN), lambda i: (i, 0)),
        pl.BlockSpec((D_IN, DS), lambda i: (0, 0)),
        pl.BlockSpec((D_IN, DS), lambda i: (0, 0)),
        pl.BlockSpec((D_IN, D_H), lambda i: (0, 0)),
        pl.BlockSpec((1, D_H), lambda i: (0, 0)),
    ],
    out_specs=[
        pl.BlockSpec((BM, DS), lambda i: (i, 0)),
        pl.BlockSpec((BM, DS), lambda i: (i, 0)),
        pl.BlockSpec((BM, D_H), lambda i: (i, 0)),
    ],
    out_shape=[
        jax.ShapeDtypeStruct((NP, DS), jnp.float32),
        jax.ShapeDtypeStruct((NP, DS), jnp.float32),
        jax.ShapeDtypeStruct((NP, D_H), jnp.float32),
    ],
)


def _tc2_body(sa0_ref, sa1_ref, sb0_ref, sb1_ref, dega_ref, degb_ref,
              pre1_ref, wl_ref, wr_ref, b2_ref, p2_ref, pre2_ref):
  deg = jnp.maximum(dega_ref[...] + degb_ref[...], 1.0)
  inv = 1.0 / deg
  aggr_lo = (sa0_ref[...] + sa1_ref[...]) * inv
  aggr_hi = (sb0_ref[...] + sb1_ref[...]) * inv
  pre1 = pre1_ref[...]
  h_lo = jnp.maximum(aggr_lo + pre1[:, :DS], 0.0)
  h_hi = jnp.maximum(aggr_hi + pre1[:, DS:], 0.0)
  h = jnp.concatenate([h_lo, h_hi], axis=1)
  p2_ref[...] = jnp.dot(h, wl_ref[...], preferred_element_type=jnp.float32)
  pre2_ref[...] = (jnp.dot(h, wr_ref[...], preferred_element_type=jnp.float32)
                   + b2_ref[...])


_tc2 = pl.pallas_call(
    _tc2_body,
    grid=(GM,),
    in_specs=[
        pl.BlockSpec((BM, DS), lambda i: (i, 0)),
        pl.BlockSpec((BM, DS), lambda i: (i, 0)),
        pl.BlockSpec((BM, DS), lambda i: (i, 0)),
        pl.BlockSpec((BM, DS), lambda i: (i, 0)),
        pl.BlockSpec((BM, 1), lambda i: (i, 0)),
        pl.BlockSpec((BM, 1), lambda i: (i, 0)),
        pl.BlockSpec((BM, D_H), lambda i: (i, 0)),
        pl.BlockSpec((D_H, D_OUT), lambda i: (0, 0)),
        pl.BlockSpec((D_H, D_OUT), lambda i: (0, 0)),
        pl.BlockSpec((1, D_OUT), lambda i: (0, 0)),
    ],
    out_specs=[
        pl.BlockSpec((BM, D_OUT), lambda i: (i, 0)),
        pl.BlockSpec((BM, D_OUT), lambda i: (i, 0)),
    ],
    out_shape=[jax.ShapeDtypeStruct((NP, D_OUT), jnp.float32)] * 2,
)


def _tc3_body(s2a_ref, s2b_ref, dega_ref, degb_ref, pre2_ref, y_ref, m_ref,
              out_ref, accs):
  i = pl.program_id(0)
  deg = jnp.maximum(dega_ref[...] + degb_ref[...], 1.0)
  z = (s2a_ref[...] + s2b_ref[...]) / deg + pre2_ref[...]
  zmax = jnp.max(z, axis=1, keepdims=True)
  lse = jnp.log(jnp.sum(jnp.exp(z - zmax), axis=1, keepdims=True)) + zmax
  logp = z - lse
  onehot = lax.broadcasted_iota(jnp.int32, z.shape, 1) == y_ref[...]
  picked = jnp.sum(jnp.where(onehot, logp, 0.0), axis=1)
  mv = m_ref[...][:, 0]
  pn = jnp.sum(picked * mv)
  pm = jnp.sum(mv)

  @pl.when(i == 0)
  def _():
    accs[0] = pn
    accs[1] = pm

  @pl.when(i > 0)
  def _():
    accs[0] += pn
    accs[1] += pm

  @pl.when(i == GM - 1)
  def _():
    out_ref[...] = jnp.full((1, 1), -accs[0] / jnp.maximum(accs[1], 1.0),
                            jnp.float32)


_tc3 = pl.pallas_call(
    _tc3_body,
    grid=(GM,),
    in_specs=[
        pl.BlockSpec((BM, D_OUT), lambda i: (i, 0)),
        pl.BlockSpec((BM, D_OUT), lambda i: (i, 0)),
        pl.BlockSpec((BM, 1), lambda i: (i, 0)),
        pl.BlockSpec((BM, 1), lambda i: (i, 0)),
        pl.BlockSpec((BM, D_OUT), lambda i: (i, 0)),
        pl.BlockSpec((BM, 1), lambda i: (i, 0)),
        pl.BlockSpec((BM, 1), lambda i: (i, 0)),
    ],
    out_specs=pl.BlockSpec((1, 1), lambda i: (0, 0)),
    out_shape=jax.ShapeDtypeStruct((1, 1), jnp.float32),
    scratch_shapes=[pltpu.SMEM((2,), jnp.float32)],
)


def kernel(x, edge_index, y, train_mask, Wl1, bl1, Wr1, br1, Wl2, bl2, Wr2,
           br2):
  src = edge_index[0]
  dst = edge_index[1]
  pad_e = E_PAD - E
  srcp = jnp.concatenate([src, jnp.zeros((pad_e,), jnp.int32)])
  srcp = srcp.reshape(TOT, CHUNK)
  dstp = jnp.concatenate([dst, jnp.full((pad_e,), N, jnp.int32)])
  dstp = dstp.reshape(TOT, CHUNK)

  xp = jnp.pad(x, ((0, NP - N), (0, 0)))
  b1 = (bl1 + br1).reshape(1, D_H)
  b2 = (bl2 + br2).reshape(1, D_OUT)

  p1a, p1b, pre1 = _tc1(xp, Wl1[:, :DS], Wl1[:, DS:], Wr1, b1)
  seg = _make_segsum(1, True)
  sa, degp = seg(p1a, srcp, dstp)
  sb, _dgb = seg(p1b, srcp, dstp)
  dega = degp[0].reshape(NP, 1)
  degb = degp[1].reshape(NP, 1)
  p2, pre2 = _tc2(sa[0], sa[1], sb[0], sb[1], dega, degb, pre1, Wl2, Wr2, b2)
  parts2, _dg2 = seg(p2, srcp, dstp)

  yp = jnp.pad(y, (0, NP - N)).reshape(NP, 1)
  mp = jnp.pad(train_mask.astype(jnp.float32), (0, NP - N)).reshape(NP, 1)
  loss = _tc3(parts2[0], parts2[1], dega, degb, pre2, yp, mp)
  return loss.reshape(1)


# speedup vs baseline: 2.4560x; 1.0081x over previous
import functools

import jax
import jax.numpy as jnp
from jax import lax
from jax.experimental import pallas as pl
from jax.experimental.pallas import tpu as pltpu
from jax.experimental.pallas import tpu_sc as plsc

N = 10000
NP = 10240
E = 320000
D_IN = 128
D_H = 128
D_OUT = 64
DS = 64

NC = 2
NS = 16
CH = 128
CHUNK = 320
TOT = 1024
E_PAD = TOT * CHUNK
K0 = 32
K1 = 32

ROWS_PT = NP // NS
SUB = ROWS_PT // CH

BM = 1280
GM = NP // BM


@functools.lru_cache(maxsize=None)
def _make_segsum(ntab, with_deg):
  mesh = plsc.VectorSubcoreMesh(core_axis_name="c", subcore_axis_name="s",
                                num_cores=NC, num_subcores=NS)
  out_type = [jax.ShapeDtypeStruct((NC, NP, DS), jnp.float32)
              for _ in range(ntab)]
  if with_deg:
    out_type.append(jax.ShapeDtypeStruct((NC, NP), jnp.float32))
  scratch = [
      pltpu.VMEM((CHUNK,), jnp.int32),
      pltpu.VMEM((CHUNK,), jnp.int32),
      pltpu.VMEM((CHUNK,), jnp.int32),
      pltpu.VMEM((CHUNK,), jnp.int32),
      pltpu.VMEM((CHUNK,), jnp.int32),
      pltpu.VMEM((CHUNK,), jnp.int32),
      pltpu.VMEM((CHUNK,), jnp.int32),
      pltpu.VMEM((CHUNK,), jnp.int32),
      pltpu.VMEM((CHUNK, DS), jnp.float32),
      pltpu.VMEM((CHUNK, DS), jnp.float32),
      pltpu.VMEM((CHUNK,), jnp.float32),
      pltpu.VMEM((CH,), jnp.float32),
      pltpu.VMEM_SHARED((NP, DS), jnp.float32),
      pltpu.VMEM_SHARED((NP, DS), jnp.float32),
  ]
  if with_deg:
    scratch.append(pltpu.VMEM_SHARED((NP,), jnp.float32))
  scratch += [pltpu.SemaphoreType.DMA, pltpu.SemaphoreType.DMA,
              pltpu.SemaphoreType.DMA]

  def body(*args):
    tabs = args[:ntab]
    srcp, dstp = args[ntab], args[ntab + 1]
    rest = args[ntab + 2:]
    outs = rest[:ntab]
    rest = rest[ntab:]
    if with_deg:
      deg_out = rest[0]
      (s0, s1, s2, s3, d0, d1, d2, d3, buf_a, buf_b, ones, vec, acc, tbl_sh,
       deg, sem_a, sem_b, sem_i) = rest[1:]
    else:
      (s0, s1, s2, s3, d0, d1, d2, d3, buf_a, buf_b, ones, vec, acc, tbl_sh,
       sem_a, sem_b, sem_i) = rest
      deg = None
      deg_out = None
    sring = (s0, s1, s2, s3)
    dring = (d0, d1, d2, d3)

    c = lax.axis_index("c")
    s = lax.axis_index("s")
    base = s * ROWS_PT
    kb = jnp.where(c == 0, K0, K1)
    rowbase = jnp.where(c == 0, s * K0, NS * K0 + s * K1)

    zf = jnp.zeros((16,), jnp.float32)

    def zero_stage():
      def zrow(i, _):
        for jj in range(DS // 16):
          buf_a[i, pl.ds(jj * 16, 16)] = zf
        return 0
      lax.fori_loop(0, CH, zrow, 0)

    if with_deg:
      for jj in range(CH // 16):
        vec[pl.ds(jj * 16, 16)] = zf
      for b in range(SUB):
        pltpu.sync_copy(vec, deg.at[pl.ds(base + b * CH, CH)])
      of = jnp.ones((16,), jnp.float32)
      for jj in range(CHUNK // 16):
        ones[pl.ds(jj * 16, 16)] = of

    def row(j):
      return rowbase + jnp.minimum(j, kb - 1)

    def i_issue(j, b):
      r = row(j)
      pltpu.async_copy(srcp.at[r], sring[b], sem_i)
      pltpu.async_copy(dstp.at[r], dring[b], sem_i)

    def i_wait(j, b):
      r = row(j)
      pltpu.make_async_copy(srcp.at[r], sring[b], sem_i).wait()
      pltpu.make_async_copy(dstp.at[r], dring[b], sem_i).wait()

    for t in range(ntab):
      tbl = tabs[t]
      out = outs[t]
      deg_on = with_deg and t == 0

      for bb in range(ROWS_PT // CH):
        off = base + bb * CH
        pltpu.sync_copy(tbl.at[pl.ds(off, CH)], buf_a.at[pl.ds(0, CH)])
        pltpu.sync_copy(buf_a.at[pl.ds(0, CH)], tbl_sh.at[pl.ds(off, CH)])

      def g_issue(b, buf, sem):
        pltpu.async_copy(tbl_sh.at[sring[b]], buf, sem)

      def g_wait(b, buf, sem):
        pltpu.make_async_copy(tbl_sh.at[sring[b]], buf, sem).wait()

      def scat(b, buf):
        pltpu.sync_copy(buf, acc.at[dring[b]], add=True)
        if deg_on:
          pltpu.sync_copy(ones, deg.at[dring[b]], add=True)

      zero_stage()
      for b in range(SUB):
        pltpu.sync_copy(buf_a.at[pl.ds(0, CH)],
                        acc.at[pl.ds(base + b * CH, CH)])
      plsc.subcore_barrier()

      i_issue(0, 0)
      i_issue(1, 1)
      i_wait(0, 0)
      g_issue(0, buf_a, sem_a)
      i_wait(1, 1)
      g_issue(1, buf_b, sem_b)
      i_issue(2, 2)
      i_issue(3, 3)

      def step(jj, _):
        j0 = 4 * jj
        for b in range(4):
          j = j0 + b
          buf, sem = (buf_a, sem_a) if b % 2 == 0 else (buf_b, sem_b)
          g_wait(b, buf, sem)
          scat(b, buf)
          i_wait(j + 2, (b + 2) % 4)
          g_issue((b + 2) % 4, buf, sem)
          i_issue(j + 4, b)
        return 0
      lax.fori_loop(0, kb // 4, step, 0)

      g_wait(0, buf_a, sem_a)
      g_wait(1, buf_b, sem_b)
      i_wait(0, 2)
      i_wait(0, 3)

      plsc.subcore_barrier()

      for b in range(SUB):
        off = base + b * CH
        pltpu.sync_copy(acc.at[pl.ds(off, CH)], buf_a.at[pl.ds(0, CH)])
        pltpu.sync_copy(buf_a.at[pl.ds(0, CH)], out.at[c, pl.ds(off, CH)])

    if with_deg:
      for b in range(SUB):
        off = base + b * CH
        pltpu.sync_copy(deg.at[pl.ds(off, CH)], vec)
        pltpu.sync_copy(vec, deg_out.at[c, pl.ds(off, CH)])

  return pl.kernel(body, out_type=tuple(out_type), mesh=mesh,
                   scratch_types=scratch,
                   compiler_params=pltpu.CompilerParams(
                       use_tc_tiling_on_sc=False))


def _tc1_body(x_ref, wla_ref, wlb_ref, wr_ref, b_ref, p1a_ref, p1b_ref,
              pre1_ref):
  xb = x_ref[...]
  p1a_ref[...] = jnp.dot(xb, wla_ref[...], preferred_element_type=jnp.float32)
  p1b_ref[...] = jnp.dot(xb, wlb_ref[...], preferred_element_type=jnp.float32)
  pre1_ref[...] = (jnp.dot(xb, wr_ref[...], preferred_element_type=jnp.float32)
                   + b_ref[...])


_tc1 = pl.pallas_call(
    _tc1_body,
    grid=(GM,),
    in_specs=[
        pl.BlockSpec((BM, D_IN), lambda i: (i, 0)),
        pl.BlockSpec((D_IN, DS), lambda i: (0, 0)),
        pl.BlockSpec((D_IN, DS), lambda i: (0, 0)),
        pl.BlockSpec((D_IN, D_H), lambda i: (0, 0)),
        pl.BlockSpec((1, D_H), lambda i: (0, 0)),
    ],
    out_specs=[
        pl.BlockSpec((BM, DS), lambda i: (i, 0)),
        pl.BlockSpec((BM, DS), lambda i: (i, 0)),
        pl.BlockSpec((BM, D_H), lambda i: (i, 0)),
    ],
    out_shape=[
        jax.ShapeDtypeStruct((NP, DS), jnp.float32),
        jax.ShapeDtypeStruct((NP, DS), jnp.float32),
        jax.ShapeDtypeStruct((NP, D_H), jnp.float32),
    ],
)


def _tc2_body(sa0_ref, sa1_ref, sb0_ref, sb1_ref, dega_ref, degb_ref,
              pre1_ref, wl_ref, wr_ref, b2_ref, p2_ref, pre2_ref):
  deg = jnp.maximum(dega_ref[...] + degb_ref[...], 1.0)
  inv = 1.0 / deg
  aggr_lo = (sa0_ref[...] + sa1_ref[...]) * inv
  aggr_hi = (sb0_ref[...] + sb1_ref[...]) * inv
  pre1 = pre1_ref[...]
  h_lo = jnp.maximum(aggr_lo + pre1[:, :DS], 0.0)
  h_hi = jnp.maximum(aggr_hi + pre1[:, DS:], 0.0)
  h = jnp.concatenate([h_lo, h_hi], axis=1)
  p2_ref[...] = jnp.dot(h, wl_ref[...], preferred_element_type=jnp.float32)
  pre2_ref[...] = (jnp.dot(h, wr_ref[...], preferred_element_type=jnp.float32)
                   + b2_ref[...])


_tc2 = pl.pallas_call(
    _tc2_body,
    grid=(GM,),
    in_specs=[
        pl.BlockSpec((BM, DS), lambda i: (i, 0)),
        pl.BlockSpec((BM, DS), lambda i: (i, 0)),
        pl.BlockSpec((BM, DS), lambda i: (i, 0)),
        pl.BlockSpec((BM, DS), lambda i: (i, 0)),
        pl.BlockSpec((BM, 1), lambda i: (i, 0)),
        pl.BlockSpec((BM, 1), lambda i: (i, 0)),
        pl.BlockSpec((BM, D_H), lambda i: (i, 0)),
        pl.BlockSpec((D_H, D_OUT), lambda i: (0, 0)),
        pl.BlockSpec((D_H, D_OUT), lambda i: (0, 0)),
        pl.BlockSpec((1, D_OUT), lambda i: (0, 0)),
    ],
    out_specs=[
        pl.BlockSpec((BM, D_OUT), lambda i: (i, 0)),
        pl.BlockSpec((BM, D_OUT), lambda i: (i, 0)),
    ],
    out_shape=[jax.ShapeDtypeStruct((NP, D_OUT), jnp.float32)] * 2,
)


def _tc3_body(s2a_ref, s2b_ref, dega_ref, degb_ref, pre2_ref, y_ref, m_ref,
              out_ref, accs):
  i = pl.program_id(0)
  deg = jnp.maximum(dega_ref[...] + degb_ref[...], 1.0)
  z = (s2a_ref[...] + s2b_ref[...]) / deg + pre2_ref[...]
  zmax = jnp.max(z, axis=1, keepdims=True)
  lse = jnp.log(jnp.sum(jnp.exp(z - zmax), axis=1, keepdims=True)) + zmax
  logp = z - lse
  onehot = lax.broadcasted_iota(jnp.int32, z.shape, 1) == y_ref[...]
  picked = jnp.sum(jnp.where(onehot, logp, 0.0), axis=1)
  mv = m_ref[...][:, 0]
  pn = jnp.sum(picked * mv)
  pm = jnp.sum(mv)

  @pl.when(i == 0)
  def _():
    accs[0] = pn
    accs[1] = pm

  @pl.when(i > 0)
  def _():
    accs[0] += pn
    accs[1] += pm

  @pl.when(i == GM - 1)
  def _():
    out_ref[...] = jnp.full((1, 1), -accs[0] / jnp.maximum(accs[1], 1.0),
                            jnp.float32)


_tc3 = pl.pallas_call(
    _tc3_body,
    grid=(GM,),
    in_specs=[
        pl.BlockSpec((BM, D_OUT), lambda i: (i, 0)),
        pl.BlockSpec((BM, D_OUT), lambda i: (i, 0)),
        pl.BlockSpec((BM, 1), lambda i: (i, 0)),
        pl.BlockSpec((BM, 1), lambda i: (i, 0)),
        pl.BlockSpec((BM, D_OUT), lambda i: (i, 0)),
        pl.BlockSpec((BM, 1), lambda i: (i, 0)),
        pl.BlockSpec((BM, 1), lambda i: (i, 0)),
    ],
    out_specs=pl.BlockSpec((1, 1), lambda i: (0, 0)),
    out_shape=jax.ShapeDtypeStruct((1, 1), jnp.float32),
    scratch_shapes=[pltpu.SMEM((2,), jnp.float32)],
)


def kernel(x, edge_index, y, train_mask, Wl1, bl1, Wr1, br1, Wl2, bl2, Wr2,
           br2):
  src = edge_index[0]
  dst = edge_index[1]
  pad_e = E_PAD - E
  srcp = jnp.concatenate([src, jnp.zeros((pad_e,), jnp.int32)])
  srcp = srcp.reshape(TOT, CHUNK)
  dstp = jnp.concatenate([dst, jnp.full((pad_e,), N, jnp.int32)])
  dstp = dstp.reshape(TOT, CHUNK)

  xp = jnp.pad(x, ((0, NP - N), (0, 0)))
  b1 = (bl1 + br1).reshape(1, D_H)
  b2 = (bl2 + br2).reshape(1, D_OUT)

  p1a, p1b, pre1 = _tc1(xp, Wl1[:, :DS], Wl1[:, DS:], Wr1, b1)
  seg = _make_segsum(1, True)
  sa, degp = seg(p1a, srcp, dstp)
  sb, _dgb = seg(p1b, srcp, dstp)
  dega = degp[0].reshape(NP, 1)
  degb = degp[1].reshape(NP, 1)
  p2, pre2 = _tc2(sa[0], sa[1], sb[0], sb[1], dega, degb, pre1, Wl2, Wr2, b2)
  parts2, _dg2 = seg(p2, srcp, dstp)

  yp = jnp.pad(y, (0, NP - N)).reshape(NP, 1)
  mp = jnp.pad(train_mask.astype(jnp.float32), (0, NP - N)).reshape(NP, 1)
  loss = _tc3(parts2[0], parts2[1], dega, degb, pre2, yp, mp)
  return loss.reshape(1)
